# baseline (speedup 1.0000x reference)
import functools

import jax
import jax.numpy as jnp
from jax import lax
from jax.experimental import pallas as pl
from jax.experimental.pallas import tpu as pltpu
from jax.experimental.pallas import tpu_sc as plsc

N_NODES = 10000
N_EDGES = 320000
D = 128

NC = 2
NS = 16
NW = NC * NS
K = 64
NCA = 160
EPT = NCA * K
E_PAD = NW * EPT
KC = 128
NCC = EPT // KC
NPAD = 10112
ROWS_PER_SUB = NPAD // NS
NRB = 4
NIB = 8


def _sc_mesh():
    return plsc.VectorSubcoreMesh(core_axis_name="c", subcore_axis_name="s")


@functools.partial(
    pl.kernel,
    out_type=(
        jax.ShapeDtypeStruct((NC, NPAD, D), jnp.float32),
        jax.ShapeDtypeStruct((NC * NPAD,), jnp.float32),
    ),
    mesh=_sc_mesh(),
    scratch_types=(
        [pltpu.VMEM_SHARED((NPAD, D), jnp.float32)]
        + [pltpu.VMEM_SHARED((NPAD,), jnp.float32)]
        + [pltpu.VMEM((K,), jnp.int32) for _ in range(NIB)]
        + [pltpu.VMEM((K,), jnp.int32) for _ in range(NIB)]
        + [pltpu.VMEM((K, D), jnp.float32) for _ in range(NRB)]
        + [pltpu.VMEM((K,), jnp.float32)]
        + [pltpu.VMEM((ROWS_PER_SUB,), jnp.float32)]
        + [pltpu.SemaphoreType.DMA for _ in range(NRB + NIB)]
    ),
    compiler_params=pltpu.CompilerParams(needs_layout_passes=False),
)
def _aggregate(feat_hbm, srcf_hbm, dstf_hbm, zrow_hbm, z1_hbm,
               agg_hbm, deg_hbm, agg_sh, deg_sh, *bufs):
    srcs = bufs[0:NIB]
    dsts = bufs[NIB:2 * NIB]
    rows = bufs[2 * NIB:2 * NIB + NRB]
    ones_v = bufs[2 * NIB + NRB]
    zbuf_v = bufs[2 * NIB + NRB + 1]
    rsem = bufs[2 * NIB + NRB + 2:2 * NIB + 2 * NRB + 2]
    isem = bufs[2 * NIB + 2 * NRB + 2:]

    c = lax.axis_index("c")
    s = lax.axis_index("s")
    wid = s * NC + c
    row0 = pl.multiple_of(s * ROWS_PER_SUB, 8)
    cbase = wid * NCA

    pltpu.sync_copy(zrow_hbm.at[pl.ds(row0, ROWS_PER_SUB)],
                    agg_sh.at[pl.ds(row0, ROWS_PER_SUB)])
    pltpu.sync_copy(z1_hbm.at[pl.ds(row0, ROWS_PER_SUB)], zbuf_v)
    pltpu.sync_copy(zbuf_v, deg_sh.at[pl.ds(row0, ROWS_PER_SUB)])
    for i in range(K // 16):
        ones_v[pl.ds(i * 16, 16)] = jnp.ones((16,), jnp.float32)
    plsc.subcore_barrier()

    def fetch(j, q):
        off = pl.multiple_of((cbase + j) * K, 8)
        pltpu.async_copy(srcf_hbm.at[pl.ds(off, K)], srcs[q], isem[q])
        pltpu.async_copy(dstf_hbm.at[pl.ds(off, K)], dsts[q], isem[q])

    def fetch_wait(j, q):
        off = pl.multiple_of((cbase + j) * K, 8)
        pltpu.make_async_copy(srcf_hbm.at[pl.ds(off, K)], srcs[q],
                              isem[q]).wait()
        pltpu.make_async_copy(dstf_hbm.at[pl.ds(off, K)], dsts[q],
                              isem[q]).wait()

    def gather(q, b):
        pltpu.async_copy(feat_hbm.at[srcs[q]], rows[b], rsem[b])

    def drain(q, b):
        pltpu.make_async_copy(feat_hbm.at[srcs[q]], rows[b], rsem[b]).wait()
        pltpu.sync_copy(rows[b], agg_sh.at[dsts[q]], add=True)
        pltpu.sync_copy(ones_v, deg_sh.at[dsts[q]], add=True)

    for q in range(NIB):
        fetch(q, q)
    for b in range(NRB):
        fetch_wait(b, b)
        gather(b, b)

    def superchunk(g, _):
        base = 8 * g
        for u in range(8):
            j = base + u
            drain(u, u % NRB)
            fetch(j + 8, u)
            fetch_wait(j + 4, (u + 4) % NIB)
            gather((u + 4) % NIB, u % NRB)
        return 0

    lax.fori_loop(0, (NCA - 8) // 8, superchunk, 0)
    for u in range(8):
        j = NCA - 8 + u
        drain(u, u % NRB)
        if u < 4:
            fetch_wait(j + 4, (u + 4) % NIB)
            gather((u + 4) % NIB, u % NRB)

    plsc.subcore_barrier()
    pltpu.sync_copy(agg_sh.at[pl.ds(row0, ROWS_PER_SUB)],
                    agg_hbm.at[c, pl.ds(row0, ROWS_PER_SUB)])
    doff = pl.multiple_of(c * NPAD + row0, 8)
    pltpu.sync_copy(deg_sh.at[pl.ds(row0, ROWS_PER_SUB)], zbuf_v)
    pltpu.sync_copy(zbuf_v, deg_hbm.at[pl.ds(doff, ROWS_PER_SUB)])


def _dense_body(feat, agg, degp, ws, wn, bc, wp, bp, s_out, t_out):
    aggs = agg[0] + agg[1]
    deg = degp[0] + degp[1]
    hn = aggs * (1.0 / jnp.maximum(deg, 1.0))[:, None]
    h = (jnp.dot(feat[...], ws[...], preferred_element_type=jnp.float32)
         + jnp.dot(hn, wn[...], preferred_element_type=jnp.float32)
         + bc[...])
    x = jnp.maximum(h, 0.0)
    b = bp[0]
    s_out[...] = jnp.sum(x * wp[0:1, :], axis=1) + b
    t_out[...] = jnp.sum(x * wp[1:2, :], axis=1) + b


def _dense(feat_pad, agg, degp, ws, wn, bc2, wp2, bp):
    return pl.pallas_call(
        _dense_body,
        out_shape=(
            jax.ShapeDtypeStruct((NPAD,), jnp.float32),
            jax.ShapeDtypeStruct((NPAD,), jnp.float32),
        ),
        in_specs=[
            pl.BlockSpec(memory_space=pltpu.VMEM),
            pl.BlockSpec(memory_space=pltpu.VMEM),
            pl.BlockSpec(memory_space=pltpu.VMEM),
            pl.BlockSpec(memory_space=pltpu.VMEM),
            pl.BlockSpec(memory_space=pltpu.VMEM),
            pl.BlockSpec(memory_space=pltpu.VMEM),
            pl.BlockSpec(memory_space=pltpu.VMEM),
            pl.BlockSpec(memory_space=pltpu.SMEM),
        ],
        out_specs=(
            pl.BlockSpec(memory_space=pltpu.VMEM),
            pl.BlockSpec(memory_space=pltpu.VMEM),
        ),
        compiler_params=pltpu.CompilerParams(
            vmem_limit_bytes=100 * 1024 * 1024,
        ),
    )(feat_pad, agg, degp, ws, wn, bc2, wp2, bp)


@functools.partial(
    pl.kernel,
    out_type=jax.ShapeDtypeStruct((NW, NCC, KC), jnp.float32),
    mesh=_sc_mesh(),
    scratch_types=[
        pltpu.VMEM((NPAD,), jnp.float32),
        pltpu.VMEM((NPAD,), jnp.float32),
        pltpu.VMEM((NCC, KC), jnp.int32),
        pltpu.VMEM((NCC, KC), jnp.int32),
        pltpu.VMEM((NCC, KC), jnp.float32),
    ],
    compiler_params=pltpu.CompilerParams(needs_layout_passes=False),
)
def _edge_logits(s_hbm, t_hbm, srcw_hbm, dstw_hbm, out_hbm,
                 s_v, t_v, src_v, dst_v, out_v):
    c = lax.axis_index("c")
    s = lax.axis_index("s")
    wid = s * NC + c

    pltpu.sync_copy(s_hbm, s_v)
    pltpu.sync_copy(t_hbm, t_v)
    pltpu.sync_copy(srcw_hbm.at[wid], src_v)
    pltpu.sync_copy(dstw_hbm.at[wid], dst_v)

    def chunk(j, _):
        for i in range(KC // 16):
            si = src_v[j, pl.ds(i * 16, 16)]
            di = dst_v[j, pl.ds(i * 16, 16)]
            z = plsc.load_gather(s_v, [si]) + plsc.load_gather(t_v, [di])
            out_v[j, pl.ds(i * 16, 16)] = 1.0 / (1.0 + jnp.exp(-z))
        return 0

    lax.fori_loop(0, NCC, chunk, 0)
    pltpu.sync_copy(out_v, out_hbm.at[wid])


def kernel(features, edge_index, edge_types, W_self, W_neigh, b_conv,
           W_pred, b_pred):
    del edge_types
    src = edge_index[0].astype(jnp.int32)
    dst = edge_index[1].astype(jnp.int32)

    pad = E_PAD - N_EDGES
    fill = jnp.full((pad,), N_NODES, jnp.int32)
    src_flat = jnp.concatenate([src, fill])
    dst_flat = jnp.concatenate([dst, fill])
    srcw = src_flat.reshape(NW, NCC, KC)
    dstw = dst_flat.reshape(NW, NCC, KC)

    feat_pad = jnp.zeros((NPAD, D), jnp.float32).at[:N_NODES].set(features)
    zrow = jnp.zeros((NPAD, D), jnp.float32)
    z1 = jnp.zeros((NPAD,), jnp.float32)

    agg, degp = _aggregate(feat_pad, src_flat, dst_flat, zrow, z1)
    degp = degp.reshape(NC, NPAD)

    wp2 = W_pred.reshape(2, D)
    bc2 = b_conv.reshape(1, D)
    s_arr, t_arr = _dense(feat_pad, agg, degp, W_self, W_neigh, bc2, wp2,
                          b_pred)

    logits = _edge_logits(s_arr, t_arr, srcw, dstw)
    return logits.reshape(-1)[:N_EDGES]


# speedup vs baseline: 1.5487x; 1.5487x over previous
import functools

import jax
import jax.numpy as jnp
from jax import lax
from jax.experimental import pallas as pl
from jax.experimental.pallas import tpu as pltpu
from jax.experimental.pallas import tpu_sc as plsc

N_NODES = 10000
N_EDGES = 320000
D = 128

NC = 2
NS = 16
NW = NC * NS
K = 128
NCHUNK = 79
EPT = NCHUNK * K
E_PAD = NW * EPT
NPAD = 10112
ROWS_PER_SUB = NPAD // NS


def _sc_mesh():
    return plsc.VectorSubcoreMesh(core_axis_name="c", subcore_axis_name="s")


@functools.partial(
    pl.kernel,
    out_type=(
        jax.ShapeDtypeStruct((NC, NPAD, D), jnp.float32),
        jax.ShapeDtypeStruct((NW * NPAD,), jnp.float32),
    ),
    mesh=_sc_mesh(),
    scratch_types=[
        pltpu.VMEM_SHARED((NPAD, D), jnp.float32),
        pltpu.VMEM((K,), jnp.int32),
        pltpu.VMEM((K,), jnp.int32),
        pltpu.VMEM((K,), jnp.int32),
        pltpu.VMEM((K,), jnp.int32),
        pltpu.VMEM((K, D), jnp.float32),
        pltpu.VMEM((K, D), jnp.float32),
        pltpu.VMEM((NPAD,), jnp.float32),
        pltpu.SemaphoreType.DMA,
        pltpu.SemaphoreType.DMA,
        pltpu.SemaphoreType.DMA,
        pltpu.SemaphoreType.DMA,
    ],
    compiler_params=pltpu.CompilerParams(needs_layout_passes=False),
)
def _aggregate(feat_hbm, srcf_hbm, dstf_hbm, zrow_hbm, z1_hbm,
               agg_hbm, deg_hbm,
               agg_sh, src0, dst0, src1, dst1, rows_a, rows_b, deg_v,
               ra, rb, si0, si1):
    c = lax.axis_index("c")
    s = lax.axis_index("s")
    wid = s * NC + c
    row0 = pl.multiple_of(s * ROWS_PER_SUB, 8)
    ebase = wid * EPT

    pltpu.sync_copy(zrow_hbm.at[pl.ds(row0, ROWS_PER_SUB)],
                    agg_sh.at[pl.ds(row0, ROWS_PER_SUB)])
    pltpu.sync_copy(z1_hbm, deg_v)
    plsc.subcore_barrier()

    ones = jnp.ones((16,), jnp.float32)

    def fetch(j, sb, db, sem):
        off = pl.multiple_of(ebase + j * K, 8)
        pltpu.async_copy(srcf_hbm.at[pl.ds(off, K)], sb, sem)
        pltpu.async_copy(dstf_hbm.at[pl.ds(off, K)], db, sem)

    def fetch_wait(j, sb, db, sem):
        off = pl.multiple_of(ebase + j * K, 8)
        pltpu.make_async_copy(srcf_hbm.at[pl.ds(off, K)], sb, sem).wait()
        pltpu.make_async_copy(dstf_hbm.at[pl.ds(off, K)], db, sem).wait()

    def gather(sb, rows, sem):
        pltpu.async_copy(feat_hbm.at[sb], rows, sem)

    def drain(db, rows, sem):
        pltpu.make_async_copy(feat_hbm.at[src0], rows, sem).wait()
        pltpu.sync_copy(rows, agg_sh.at[db], add=True)
        for i in range(K // 16):
            plsc.addupdate_scatter(deg_v, [db[pl.ds(i * 16, 16)]], ones)

    fetch(0, src0, dst0, si0)
    fetch_wait(0, src0, dst0, si0)
    gather(src0, rows_a, ra)
    fetch(1, src1, dst1, si1)

    def two_chunks(g, _):
        j = 2 * g
        fetch_wait(j + 1, src1, dst1, si1)
        gather(src1, rows_b, rb)
        drain(dst0, rows_a, ra)
        fetch(j + 2, src0, dst0, si0)
        fetch_wait(j + 2, src0, dst0, si0)
        gather(src0, rows_a, ra)
        drain(dst1, rows_b, rb)
        fetch(j + 3, src1, dst1, si1)
        return 0

    lax.fori_loop(0, (NCHUNK - 3) // 2, two_chunks, 0)
    fetch_wait(NCHUNK - 2, src1, dst1, si1)
    gather(src1, rows_b, rb)
    drain(dst0, rows_a, ra)
    fetch(NCHUNK - 1, src0, dst0, si0)
    fetch_wait(NCHUNK - 1, src0, dst0, si0)
    gather(src0, rows_a, ra)
    drain(dst1, rows_b, rb)
    drain(dst0, rows_a, ra)

    doff = pl.multiple_of(wid * NPAD, 8)
    pltpu.sync_copy(deg_v, deg_hbm.at[pl.ds(doff, NPAD)])
    plsc.subcore_barrier()
    pltpu.sync_copy(agg_sh.at[pl.ds(row0, ROWS_PER_SUB)],
                    agg_hbm.at[c, pl.ds(row0, ROWS_PER_SUB)])


def _dense_body(feat, agg, degp, ws, wn, bc, wp, bp, s_out, t_out):
    aggs = agg[0] + agg[1]
    deg = jnp.sum(degp[...], axis=0)
    hn = aggs * (1.0 / jnp.maximum(deg, 1.0))[:, None]
    h = (jnp.dot(feat[...], ws[...], preferred_element_type=jnp.float32)
         + jnp.dot(hn, wn[...], preferred_element_type=jnp.float32)
         + bc[...])
    x = jnp.maximum(h, 0.0)
    b = bp[0]
    s_out[...] = jnp.sum(x * wp[0:1, :], axis=1) + b
    t_out[...] = jnp.sum(x * wp[1:2, :], axis=1) + b


def _dense(feat_pad, agg, degp, ws, wn, bc2, wp2, bp):
    return pl.pallas_call(
        _dense_body,
        out_shape=(
            jax.ShapeDtypeStruct((NPAD,), jnp.float32),
            jax.ShapeDtypeStruct((NPAD,), jnp.float32),
        ),
        in_specs=[
            pl.BlockSpec(memory_space=pltpu.VMEM),
            pl.BlockSpec(memory_space=pltpu.VMEM),
            pl.BlockSpec(memory_space=pltpu.VMEM),
            pl.BlockSpec(memory_space=pltpu.VMEM),
            pl.BlockSpec(memory_space=pltpu.VMEM),
            pl.BlockSpec(memory_space=pltpu.VMEM),
            pl.BlockSpec(memory_space=pltpu.VMEM),
            pl.BlockSpec(memory_space=pltpu.SMEM),
        ],
        out_specs=(
            pl.BlockSpec(memory_space=pltpu.VMEM),
            pl.BlockSpec(memory_space=pltpu.VMEM),
        ),
        compiler_params=pltpu.CompilerParams(
            vmem_limit_bytes=100 * 1024 * 1024,
        ),
    )(feat_pad, agg, degp, ws, wn, bc2, wp2, bp)


@functools.partial(
    pl.kernel,
    out_type=jax.ShapeDtypeStruct((NW, NCHUNK, K), jnp.float32),
    mesh=_sc_mesh(),
    scratch_types=[
        pltpu.VMEM((NPAD,), jnp.float32),
        pltpu.VMEM((NPAD,), jnp.float32),
        pltpu.VMEM((NCHUNK, K), jnp.int32),
        pltpu.VMEM((NCHUNK, K), jnp.int32),
        pltpu.VMEM((NCHUNK, K), jnp.float32),
    ],
    compiler_params=pltpu.CompilerParams(needs_layout_passes=False),
)
def _edge_logits(s_hbm, t_hbm, srcw_hbm, dstw_hbm, out_hbm,
                 s_v, t_v, src_v, dst_v, out_v):
    c = lax.axis_index("c")
    s = lax.axis_index("s")
    wid = s * NC + c

    pltpu.sync_copy(s_hbm, s_v)
    pltpu.sync_copy(t_hbm, t_v)
    pltpu.sync_copy(srcw_hbm.at[wid], src_v)
    pltpu.sync_copy(dstw_hbm.at[wid], dst_v)

    def chunk(j, _):
        for i in range(K // 16):
            si = src_v[j, pl.ds(i * 16, 16)]
            di = dst_v[j, pl.ds(i * 16, 16)]
            z = plsc.load_gather(s_v, [si]) + plsc.load_gather(t_v, [di])
            out_v[j, pl.ds(i * 16, 16)] = 1.0 / (1.0 + jnp.exp(-z))
        return 0

    lax.fori_loop(0, NCHUNK, chunk, 0)
    pltpu.sync_copy(out_v, out_hbm.at[wid])


def kernel(features, edge_index, edge_types, W_self, W_neigh, b_conv,
           W_pred, b_pred):
    del edge_types
    src = edge_index[0].astype(jnp.int32)
    dst = edge_index[1].astype(jnp.int32)

    pad = E_PAD - N_EDGES
    fill = jnp.full((pad,), N_NODES, jnp.int32)
    src_flat = jnp.concatenate([src, fill])
    dst_flat = jnp.concatenate([dst, fill])
    srcw = src_flat.reshape(NW, NCHUNK, K)
    dstw = dst_flat.reshape(NW, NCHUNK, K)

    feat_pad = jnp.zeros((NPAD, D), jnp.float32).at[:N_NODES].set(features)
    zrow = jnp.zeros((NPAD, D), jnp.float32)
    z1 = jnp.zeros((NPAD,), jnp.float32)

    agg, degp = _aggregate(feat_pad, src_flat, dst_flat, zrow, z1)
    degp = degp.reshape(NW, NPAD)

    wp2 = W_pred.reshape(2, D)
    bc2 = b_conv.reshape(1, D)
    s_arr, t_arr = _dense(feat_pad, agg, degp, W_self, W_neigh, bc2, wp2,
                          b_pred)

    logits = _edge_logits(s_arr, t_arr, srcw, dstw)
    return logits.reshape(-1)[:N_EDGES]


# speedup vs baseline: 1.5800x; 1.0202x over previous
import functools

import jax
import jax.numpy as jnp
from jax import lax
from jax.experimental import pallas as pl
from jax.experimental.pallas import tpu as pltpu
from jax.experimental.pallas import tpu_sc as plsc

N_NODES = 10000
N_EDGES = 320000
D = 128

NC = 2
NS = 16
NW = NC * NS
K = 128
NCHUNK = 79
EPT = NCHUNK * K
E_PAD = NW * EPT
NPAD = 10112
ROWS_PER_SUB = NPAD // NS


def _sc_mesh():
    return plsc.VectorSubcoreMesh(core_axis_name="c", subcore_axis_name="s")


@functools.partial(
    pl.kernel,
    out_type=(
        jax.ShapeDtypeStruct((NC, NPAD, D), jnp.float32),
        jax.ShapeDtypeStruct((NW * NPAD,), jnp.float32),
    ),
    mesh=_sc_mesh(),
    scratch_types=[
        pltpu.VMEM_SHARED((NPAD, D), jnp.float32),
        pltpu.VMEM((K,), jnp.int32),
        pltpu.VMEM((K,), jnp.int32),
        pltpu.VMEM((K,), jnp.int32),
        pltpu.VMEM((K,), jnp.int32),
        pltpu.VMEM((K,), jnp.int32),
        pltpu.VMEM((K,), jnp.int32),
        pltpu.VMEM((K,), jnp.int32),
        pltpu.VMEM((K,), jnp.int32),
        pltpu.VMEM((K, D), jnp.float32),
        pltpu.VMEM((K, D), jnp.float32),
        pltpu.VMEM((NPAD,), jnp.float32),
        pltpu.SemaphoreType.DMA,
        pltpu.SemaphoreType.DMA,
        pltpu.SemaphoreType.DMA,
        pltpu.SemaphoreType.DMA,
        pltpu.SemaphoreType.DMA,
        pltpu.SemaphoreType.DMA,
    ],
    compiler_params=pltpu.CompilerParams(needs_layout_passes=False),
)
def _aggregate(feat_hbm, srcf_hbm, dstf_hbm, zrow_hbm, z1_hbm,
               agg_hbm, deg_hbm,
               agg_sh, s0, s1, s2, s3, d0, d1, d2, d3, rows_a, rows_b,
               deg_v, ra, rb, i0, i1, i2, i3):
    srcs = (s0, s1, s2, s3)
    dsts = (d0, d1, d2, d3)
    isem = (i0, i1, i2, i3)
    c = lax.axis_index("c")
    s = lax.axis_index("s")
    wid = s * NC + c
    row0 = pl.multiple_of(s * ROWS_PER_SUB, 8)
    ebase = wid * EPT

    pltpu.sync_copy(zrow_hbm.at[pl.ds(row0, ROWS_PER_SUB)],
                    agg_sh.at[pl.ds(row0, ROWS_PER_SUB)])
    pltpu.sync_copy(z1_hbm, deg_v)
    plsc.subcore_barrier()

    ones = jnp.ones((16,), jnp.float32)

    def fetch(j, q):
        off = pl.multiple_of(ebase + j * K, 8)
        pltpu.async_copy(srcf_hbm.at[pl.ds(off, K)], srcs[q], isem[q])
        pltpu.async_copy(dstf_hbm.at[pl.ds(off, K)], dsts[q], isem[q])

    def fetch_wait(j, q):
        off = pl.multiple_of(ebase + j * K, 8)
        pltpu.make_async_copy(srcf_hbm.at[pl.ds(off, K)], srcs[q],
                              isem[q]).wait()
        pltpu.make_async_copy(dstf_hbm.at[pl.ds(off, K)], dsts[q],
                              isem[q]).wait()

    def gather(q, rows, sem):
        pltpu.async_copy(feat_hbm.at[srcs[q]], rows, sem)

    def drain(q, rows, sem):
        pltpu.make_async_copy(feat_hbm.at[srcs[0]], rows, sem).wait()
        pltpu.sync_copy(rows, agg_sh.at[dsts[q]], add=True)
        for i in range(K // 16):
            plsc.addupdate_scatter(deg_v, [dsts[q][pl.ds(i * 16, 16)]],
                                   ones)

    for q in range(4):
        fetch(q, q)
    fetch_wait(0, 0)
    gather(0, rows_a, ra)
    fetch_wait(1, 1)
    gather(1, rows_b, rb)

    def four_chunks(g, _):
        j = 4 * g
        fetch_wait(j + 2, 2)
        drain(0, rows_a, ra)
        fetch(j + 4, 0)
        gather(2, rows_a, ra)
        fetch_wait(j + 3, 3)
        drain(1, rows_b, rb)
        fetch(j + 5, 1)
        gather(3, rows_b, rb)
        fetch_wait(j + 4, 0)
        drain(2, rows_a, ra)
        fetch(j + 6, 2)
        gather(0, rows_a, ra)
        fetch_wait(j + 5, 1)
        drain(3, rows_b, rb)
        fetch(j + 7, 3)
        gather(1, rows_b, rb)
        return 0

    lax.fori_loop(0, (NCHUNK - 7) // 4, four_chunks, 0)
    n7 = NCHUNK - 7
    fetch_wait(n7 + 2, 2)
    drain(0, rows_a, ra)
    fetch(n7 + 4, 0)
    gather(2, rows_a, ra)
    fetch_wait(n7 + 3, 3)
    drain(1, rows_b, rb)
    fetch(n7 + 5, 1)
    gather(3, rows_b, rb)
    fetch_wait(n7 + 4, 0)
    drain(2, rows_a, ra)
    fetch(n7 + 6, 2)
    gather(0, rows_a, ra)
    fetch_wait(n7 + 5, 1)
    drain(3, rows_b, rb)
    gather(1, rows_b, rb)
    fetch_wait(n7 + 6, 2)
    drain(0, rows_a, ra)
    gather(2, rows_a, ra)
    drain(1, rows_b, rb)
    drain(2, rows_a, ra)

    doff = pl.multiple_of(wid * NPAD, 8)
    pltpu.sync_copy(deg_v, deg_hbm.at[pl.ds(doff, NPAD)])
    plsc.subcore_barrier()
    pltpu.sync_copy(agg_sh.at[pl.ds(row0, ROWS_PER_SUB)],
                    agg_hbm.at[c, pl.ds(row0, ROWS_PER_SUB)])


def _dense_body(feat, agg, degp, ws, wn, bc, wp, bp, s_out, t_out):
    aggs = agg[0] + agg[1]
    deg = jnp.sum(degp[...], axis=0)
    hn = aggs * (1.0 / jnp.maximum(deg, 1.0))[:, None]
    h = (jnp.dot(feat[...], ws[...], preferred_element_type=jnp.float32)
         + jnp.dot(hn, wn[...], preferred_element_type=jnp.float32)
         + bc[...])
    x = jnp.maximum(h, 0.0)
    b = bp[0]
    s_out[...] = jnp.sum(x * wp[0:1, :], axis=1) + b
    t_out[...] = jnp.sum(x * wp[1:2, :], axis=1) + b


def _dense(feat_pad, agg, degp, ws, wn, bc2, wp2, bp):
    return pl.pallas_call(
        _dense_body,
        out_shape=(
            jax.ShapeDtypeStruct((NPAD,), jnp.float32),
            jax.ShapeDtypeStruct((NPAD,), jnp.float32),
        ),
        in_specs=[
            pl.BlockSpec(memory_space=pltpu.VMEM),
            pl.BlockSpec(memory_space=pltpu.VMEM),
            pl.BlockSpec(memory_space=pltpu.VMEM),
            pl.BlockSpec(memory_space=pltpu.VMEM),
            pl.BlockSpec(memory_space=pltpu.VMEM),
            pl.BlockSpec(memory_space=pltpu.VMEM),
            pl.BlockSpec(memory_space=pltpu.VMEM),
            pl.BlockSpec(memory_space=pltpu.SMEM),
        ],
        out_specs=(
            pl.BlockSpec(memory_space=pltpu.VMEM),
            pl.BlockSpec(memory_space=pltpu.VMEM),
        ),
        compiler_params=pltpu.CompilerParams(
            vmem_limit_bytes=100 * 1024 * 1024,
        ),
    )(feat_pad, agg, degp, ws, wn, bc2, wp2, bp)


@functools.partial(
    pl.kernel,
    out_type=jax.ShapeDtypeStruct((NW, NCHUNK, K), jnp.float32),
    mesh=_sc_mesh(),
    scratch_types=[
        pltpu.VMEM((NPAD,), jnp.float32),
        pltpu.VMEM((NPAD,), jnp.float32),
        pltpu.VMEM((NCHUNK, K), jnp.int32),
        pltpu.VMEM((NCHUNK, K), jnp.int32),
        pltpu.VMEM((NCHUNK, K), jnp.float32),
    ],
    compiler_params=pltpu.CompilerParams(needs_layout_passes=False),
)
def _edge_logits(s_hbm, t_hbm, srcw_hbm, dstw_hbm, out_hbm,
                 s_v, t_v, src_v, dst_v, out_v):
    c = lax.axis_index("c")
    s = lax.axis_index("s")
    wid = s * NC + c

    pltpu.sync_copy(s_hbm, s_v)
    pltpu.sync_copy(t_hbm, t_v)
    pltpu.sync_copy(srcw_hbm.at[wid], src_v)
    pltpu.sync_copy(dstw_hbm.at[wid], dst_v)

    def chunk(j, _):
        for i in range(K // 16):
            si = src_v[j, pl.ds(i * 16, 16)]
            di = dst_v[j, pl.ds(i * 16, 16)]
            z = plsc.load_gather(s_v, [si]) + plsc.load_gather(t_v, [di])
            out_v[j, pl.ds(i * 16, 16)] = 1.0 / (1.0 + jnp.exp(-z))
        return 0

    lax.fori_loop(0, NCHUNK, chunk, 0)
    pltpu.sync_copy(out_v, out_hbm.at[wid])


def kernel(features, edge_index, edge_types, W_self, W_neigh, b_conv,
           W_pred, b_pred):
    del edge_types
    src = edge_index[0].astype(jnp.int32)
    dst = edge_index[1].astype(jnp.int32)

    pad = E_PAD - N_EDGES
    fill = jnp.full((pad,), N_NODES, jnp.int32)
    src_flat = jnp.concatenate([src, fill])
    dst_flat = jnp.concatenate([dst, fill])
    srcw = src_flat.reshape(NW, NCHUNK, K)
    dstw = dst_flat.reshape(NW, NCHUNK, K)

    feat_pad = jnp.zeros((NPAD, D), jnp.float32).at[:N_NODES].set(features)
    zrow = jnp.zeros((NPAD, D), jnp.float32)
    z1 = jnp.zeros((NPAD,), jnp.float32)

    agg, degp = _aggregate(feat_pad, src_flat, dst_flat, zrow, z1)
    degp = degp.reshape(NW, NPAD)

    wp2 = W_pred.reshape(2, D)
    bc2 = b_conv.reshape(1, D)
    s_arr, t_arr = _dense(feat_pad, agg, degp, W_self, W_neigh, bc2, wp2,
                          b_pred)

    logits = _edge_logits(s_arr, t_arr, srcw, dstw)
    return logits.reshape(-1)[:N_EDGES]


# speedup vs baseline: 1.5865x; 1.0041x over previous
import functools

import jax
import jax.numpy as jnp
from jax import lax
from jax.experimental import pallas as pl
from jax.experimental.pallas import tpu as pltpu
from jax.experimental.pallas import tpu_sc as plsc

N_NODES = 10000
N_EDGES = 320000
D = 128

NC = 2
NS = 16
NW = NC * NS
K = 128
NCHUNK = 79
EPT = NCHUNK * K
E_PAD = NW * EPT
NPAD = 10112
ROWS_PER_SUB = NPAD // NS


def _sc_mesh():
    return plsc.VectorSubcoreMesh(core_axis_name="c", subcore_axis_name="s")


@functools.partial(
    pl.kernel,
    out_type=(
        jax.ShapeDtypeStruct((NC, NPAD, D), jnp.float32),
        jax.ShapeDtypeStruct((NW * NPAD,), jnp.float32),
    ),
    mesh=_sc_mesh(),
    scratch_types=[
        pltpu.VMEM_SHARED((NPAD, D), jnp.float32),
        pltpu.VMEM((K,), jnp.int32),
        pltpu.VMEM((K,), jnp.int32),
        pltpu.VMEM((K,), jnp.int32),
        pltpu.VMEM((K,), jnp.int32),
        pltpu.VMEM((K,), jnp.int32),
        pltpu.VMEM((K,), jnp.int32),
        pltpu.VMEM((K,), jnp.int32),
        pltpu.VMEM((K,), jnp.int32),
        pltpu.VMEM((K, D), jnp.float32),
        pltpu.VMEM((K, D), jnp.float32),
        pltpu.VMEM((NPAD,), jnp.float32),
        pltpu.SemaphoreType.DMA,
        pltpu.SemaphoreType.DMA,
        pltpu.SemaphoreType.DMA,
        pltpu.SemaphoreType.DMA,
        pltpu.SemaphoreType.DMA,
        pltpu.SemaphoreType.DMA,
    ],
    compiler_params=pltpu.CompilerParams(needs_layout_passes=False),
)
def _aggregate(feat_hbm, srcf_hbm, dstf_hbm, z1_hbm,
               agg_hbm, deg_hbm,
               agg_sh, s0, s1, s2, s3, d0, d1, d2, d3, rows_a, rows_b,
               deg_v, ra, rb, i0, i1, i2, i3):
    srcs = (s0, s1, s2, s3)
    dsts = (d0, d1, d2, d3)
    isem = (i0, i1, i2, i3)
    c = lax.axis_index("c")
    s = lax.axis_index("s")
    wid = s * NC + c
    row0 = pl.multiple_of(s * ROWS_PER_SUB, 8)
    ebase = wid * EPT

    pltpu.sync_copy(z1_hbm, deg_v)
    zv = jnp.zeros((16,), jnp.float32)

    def zrow_body(r, _):
        for i in range(D // 16):
            rows_a[r, pl.ds(i * 16, 16)] = zv
        return 0

    lax.fori_loop(0, K, zrow_body, 0)
    for kk in range(ROWS_PER_SUB // K):
        pltpu.sync_copy(rows_a, agg_sh.at[pl.ds(row0 + kk * K, K)])
    rem = ROWS_PER_SUB % K
    pltpu.sync_copy(
        rows_a.at[pl.ds(0, rem)],
        agg_sh.at[pl.ds(row0 + (ROWS_PER_SUB // K) * K, rem)])
    plsc.subcore_barrier()

    ones = jnp.ones((16,), jnp.float32)

    def fetch(j, q):
        off = pl.multiple_of(ebase + j * K, 8)
        pltpu.async_copy(srcf_hbm.at[pl.ds(off, K)], srcs[q], isem[q])
        pltpu.async_copy(dstf_hbm.at[pl.ds(off, K)], dsts[q], isem[q])

    def fetch_wait(j, q):
        off = pl.multiple_of(ebase + j * K, 8)
        pltpu.make_async_copy(srcf_hbm.at[pl.ds(off, K)], srcs[q],
                              isem[q]).wait()
        pltpu.make_async_copy(dstf_hbm.at[pl.ds(off, K)], dsts[q],
                              isem[q]).wait()

    def gather(q, rows, sem):
        pltpu.async_copy(feat_hbm.at[srcs[q]], rows, sem)

    def drain(q, rows, sem):
        pltpu.make_async_copy(feat_hbm.at[srcs[0]], rows, sem).wait()
        pltpu.sync_copy(rows, agg_sh.at[dsts[q]], add=True)
        for i in range(K // 16):
            plsc.addupdate_scatter(deg_v, [dsts[q][pl.ds(i * 16, 16)]],
                                   ones)

    for q in range(4):
        fetch(q, q)
    fetch_wait(0, 0)
    gather(0, rows_a, ra)
    fetch_wait(1, 1)
    gather(1, rows_b, rb)

    def four_chunks(g, _):
        j = 4 * g
        fetch_wait(j + 2, 2)
        drain(0, rows_a, ra)
        fetch(j + 4, 0)
        gather(2, rows_a, ra)
        fetch_wait(j + 3, 3)
        drain(1, rows_b, rb)
        fetch(j + 5, 1)
        gather(3, rows_b, rb)
        fetch_wait(j + 4, 0)
        drain(2, rows_a, ra)
        fetch(j + 6, 2)
        gather(0, rows_a, ra)
        fetch_wait(j + 5, 1)
        drain(3, rows_b, rb)
        fetch(j + 7, 3)
        gather(1, rows_b, rb)
        return 0

    lax.fori_loop(0, (NCHUNK - 7) // 4, four_chunks, 0)
    n7 = NCHUNK - 7
    fetch_wait(n7 + 2, 2)
    drain(0, rows_a, ra)
    fetch(n7 + 4, 0)
    gather(2, rows_a, ra)
    fetch_wait(n7 + 3, 3)
    drain(1, rows_b, rb)
    fetch(n7 + 5, 1)
    gather(3, rows_b, rb)
    fetch_wait(n7 + 4, 0)
    drain(2, rows_a, ra)
    fetch(n7 + 6, 2)
    gather(0, rows_a, ra)
    fetch_wait(n7 + 5, 1)
    drain(3, rows_b, rb)
    gather(1, rows_b, rb)
    fetch_wait(n7 + 6, 2)
    drain(0, rows_a, ra)
    gather(2, rows_a, ra)
    drain(1, rows_b, rb)
    drain(2, rows_a, ra)

    doff = pl.multiple_of(wid * NPAD, 8)
    pltpu.sync_copy(deg_v, deg_hbm.at[pl.ds(doff, NPAD)])
    plsc.subcore_barrier()
    pltpu.sync_copy(agg_sh.at[pl.ds(row0, ROWS_PER_SUB)],
                    agg_hbm.at[c, pl.ds(row0, ROWS_PER_SUB)])


def _dense_body(feat, agg, degp, ws, wn, bc, wp, bp, s_out, t_out):
    aggs = agg[0] + agg[1]
    deg = jnp.sum(degp[...], axis=0)
    hn = aggs * (1.0 / jnp.maximum(deg, 1.0))[:, None]
    h = (jnp.dot(feat[...], ws[...], preferred_element_type=jnp.float32)
         + jnp.dot(hn, wn[...], preferred_element_type=jnp.float32)
         + bc[...])
    x = jnp.maximum(h, 0.0)
    b = bp[0]
    s_out[...] = jnp.sum(x * wp[0:1, :], axis=1) + b
    t_out[...] = jnp.sum(x * wp[1:2, :], axis=1) + b


def _dense(feat_pad, agg, degp, ws, wn, bc2, wp2, bp):
    return pl.pallas_call(
        _dense_body,
        out_shape=(
            jax.ShapeDtypeStruct((NPAD,), jnp.float32),
            jax.ShapeDtypeStruct((NPAD,), jnp.float32),
        ),
        in_specs=[
            pl.BlockSpec(memory_space=pltpu.VMEM),
            pl.BlockSpec(memory_space=pltpu.VMEM),
            pl.BlockSpec(memory_space=pltpu.VMEM),
            pl.BlockSpec(memory_space=pltpu.VMEM),
            pl.BlockSpec(memory_space=pltpu.VMEM),
            pl.BlockSpec(memory_space=pltpu.VMEM),
            pl.BlockSpec(memory_space=pltpu.VMEM),
            pl.BlockSpec(memory_space=pltpu.SMEM),
        ],
        out_specs=(
            pl.BlockSpec(memory_space=pltpu.VMEM),
            pl.BlockSpec(memory_space=pltpu.VMEM),
        ),
        compiler_params=pltpu.CompilerParams(
            vmem_limit_bytes=100 * 1024 * 1024,
        ),
    )(feat_pad, agg, degp, ws, wn, bc2, wp2, bp)


@functools.partial(
    pl.kernel,
    out_type=jax.ShapeDtypeStruct((NW, NCHUNK, K), jnp.float32),
    mesh=_sc_mesh(),
    scratch_types=[
        pltpu.VMEM((NPAD,), jnp.float32),
        pltpu.VMEM((NPAD,), jnp.float32),
        pltpu.VMEM((NCHUNK, K), jnp.int32),
        pltpu.VMEM((NCHUNK, K), jnp.int32),
        pltpu.VMEM((NCHUNK, K), jnp.float32),
    ],
    compiler_params=pltpu.CompilerParams(needs_layout_passes=False),
)
def _edge_logits(s_hbm, t_hbm, srcw_hbm, dstw_hbm, out_hbm,
                 s_v, t_v, src_v, dst_v, out_v):
    c = lax.axis_index("c")
    s = lax.axis_index("s")
    wid = s * NC + c

    pltpu.sync_copy(s_hbm, s_v)
    pltpu.sync_copy(t_hbm, t_v)
    pltpu.sync_copy(srcw_hbm.at[wid], src_v)
    pltpu.sync_copy(dstw_hbm.at[wid], dst_v)

    def chunk(j, _):
        for i in range(K // 16):
            si = src_v[j, pl.ds(i * 16, 16)]
            di = dst_v[j, pl.ds(i * 16, 16)]
            z = plsc.load_gather(s_v, [si]) + plsc.load_gather(t_v, [di])
            out_v[j, pl.ds(i * 16, 16)] = 1.0 / (1.0 + jnp.exp(-z))
        return 0

    lax.fori_loop(0, NCHUNK, chunk, 0)
    pltpu.sync_copy(out_v, out_hbm.at[wid])


def kernel(features, edge_index, edge_types, W_self, W_neigh, b_conv,
           W_pred, b_pred):
    del edge_types
    src = edge_index[0].astype(jnp.int32)
    dst = edge_index[1].astype(jnp.int32)

    pad = E_PAD - N_EDGES
    fill = jnp.full((pad,), N_NODES, jnp.int32)
    src_flat = jnp.concatenate([src, fill])
    dst_flat = jnp.concatenate([dst, fill])
    srcw = src_flat.reshape(NW, NCHUNK, K)
    dstw = dst_flat.reshape(NW, NCHUNK, K)

    feat_pad = jnp.zeros((NPAD, D), jnp.float32).at[:N_NODES].set(features)
    z1 = jnp.zeros((NPAD,), jnp.float32)

    agg, degp = _aggregate(feat_pad, src_flat, dst_flat, z1)
    degp = degp.reshape(NW, NPAD)

    wp2 = W_pred.reshape(2, D)
    bc2 = b_conv.reshape(1, D)
    s_arr, t_arr = _dense(feat_pad, agg, degp, W_self, W_neigh, bc2, wp2,
                          b_pred)

    logits = _edge_logits(s_arr, t_arr, srcw, dstw)
    return logits.reshape(-1)[:N_EDGES]
